# baseline (speedup 1.0000x reference)
import functools

import jax
import jax.numpy as jnp
from jax import lax
from jax.experimental import pallas as pl
from jax.experimental.pallas import tpu as pltpu
from jax.experimental.pallas import tpu_sc as plsc

U = 100000
I = 100000
E = 64
B = 16384

CHUNK = 128
NUM_CHUNKS = B // CHUNK
TBLK = 4096
SBLK = 2048

_TLHS = (((0,), (0,)), ((), ()))
_TLHS_RT = (((0,), (1,)), ((), ()))


def _user_body(wut, c0t, c1t, c2t, wit, w_dnn_t, b_dnn, w_di, b_di,
               wd0, bd0, wd1, bd1, wd2, bd2, u4_ref, ipk_ref):
  f32 = jnp.float32
  pre = b_dnn[...]
  for k, t in enumerate((wut, c0t, c1t, c2t)):
    pre = pre + lax.dot_general(t[...], w_dnn_t[:, k * E:(k + 1) * E],
                                _TLHS_RT, preferred_element_type=f32)
  server = jax.nn.relu(pre)
  d0 = jax.nn.relu(
      lax.dot_general(c0t[...], wd0[...], _TLHS,
                      preferred_element_type=f32) + bd0[...])
  d1 = jax.nn.relu(
      lax.dot_general(c1t[...], wd1[...], _TLHS,
                      preferred_element_type=f32) + bd1[...])
  d2 = jax.nn.relu(
      lax.dot_general(c2t[...], wd2[...], _TLHS,
                      preferred_element_type=f32) + bd2[...])

  def pack(a, b):
    a32 = lax.bitcast_convert_type(
        a.astype(jnp.bfloat16), jnp.uint16).astype(jnp.uint32)
    b32 = lax.bitcast_convert_type(
        b.astype(jnp.bfloat16), jnp.uint16).astype(jnp.uint32)
    return a32 | (b32 << 16)

  u4_ref[:, 0:E] = pack(server, d0)
  u4_ref[:, E:2 * E] = pack(d1, d2)
  e = jax.nn.relu(
      lax.dot_general(wit[...], w_di[...], _TLHS,
                      preferred_element_type=f32) + b_di[...])
  ipk_ref[:, 0:E] = e
  ipk_ref[:, E:2 * E] = e


def _transform(w_user, w_item, c0, c1, c2, w_dnn, b_dnn, w_di, b_di,
               wd0, bd0, wd1, bd1, wd2, bd2):
  grid = (pl.cdiv(U, TBLK),)
  t_spec = pl.BlockSpec((E, TBLK), lambda i: (0, i))
  full = lambda shape: pl.BlockSpec(shape, lambda i: (0,) * len(shape))
  return pl.pallas_call(
      _user_body,
      grid=grid,
      in_specs=[t_spec] * 5 + [
          full((E, 4 * E)), full((1, E)),
          full((E, E)), full((1, E)),
          full((E, E)), full((1, E)),
          full((E, E)), full((1, E)),
          full((E, E)), full((1, E)),
      ],
      out_specs=[
          pl.BlockSpec((TBLK, 2 * E), lambda i: (i, 0)),
          pl.BlockSpec((TBLK, 2 * E), lambda i: (i, 0)),
      ],
      out_shape=[
          jax.ShapeDtypeStruct((U, 2 * E), jnp.uint32),
          jax.ShapeDtypeStruct((I, 2 * E), jnp.float32),
      ],
      compiler_params=pltpu.CompilerParams(
          fuse_transposed_lhs_in_matmul=True),
  )(w_user.T, c0.T, c1.T, c2.T, w_item.T, w_dnn.T, b_dnn.reshape(1, E),
    w_di, b_di.reshape(1, E), wd0, bd0.reshape(1, E),
    wd1, bd1.reshape(1, E), wd2, bd2.reshape(1, E))


def _gather_body(nchunks_per_worker, num_cores,
                 users_hbm, pos_hbm, neg_hbm, u4, ipk,
                 out_u, out_p, out_n,
                 idx_v, urows_v, irows_v, sem):
  wid = lax.axis_index("s") * num_cores + lax.axis_index("c")
  row0 = wid * nchunks_per_worker
  for j in range(nchunks_per_worker):
    crow = row0 + j
    base = crow * CHUNK
    pltpu.sync_copy(users_hbm.at[crow], idx_v)
    pltpu.async_copy(u4.at[idx_v], urows_v, sem).wait()
    pltpu.sync_copy(urows_v, out_u.at[pl.ds(base, CHUNK)])
    pltpu.sync_copy(pos_hbm.at[crow], idx_v)
    pltpu.async_copy(ipk.at[idx_v], irows_v, sem).wait()
    pltpu.sync_copy(irows_v, out_p.at[pl.ds(base, CHUNK)])
    pltpu.sync_copy(neg_hbm.at[crow], idx_v)
    pltpu.async_copy(ipk.at[idx_v], irows_v, sem).wait()
    pltpu.sync_copy(irows_v, out_n.at[pl.ds(base, CHUNK)])


def _sc_gather(users, pos_items, neg_items, u4, ipk):
  info = plsc.get_sparse_core_info()
  num_cores, num_subcores = info.num_cores, info.num_subcores
  npw = NUM_CHUNKS // (num_cores * num_subcores)
  mesh = plsc.VectorSubcoreMesh(core_axis_name="c", subcore_axis_name="s")
  out_t = [
      jax.ShapeDtypeStruct((B, 2 * E), jnp.uint32),
      jax.ShapeDtypeStruct((B, 2 * E), jnp.float32),
      jax.ShapeDtypeStruct((B, 2 * E), jnp.float32),
  ]
  scratch = [
      pltpu.VMEM((CHUNK,), jnp.int32),
      pltpu.VMEM((CHUNK, 2 * E), jnp.uint32),
      pltpu.VMEM((CHUNK, 2 * E), jnp.float32),
      pltpu.SemaphoreType.DMA,
  ]
  users2 = users.astype(jnp.int32).reshape(NUM_CHUNKS, CHUNK)
  pos2 = pos_items.astype(jnp.int32).reshape(NUM_CHUNKS, CHUNK)
  neg2 = neg_items.astype(jnp.int32).reshape(NUM_CHUNKS, CHUNK)
  body = functools.partial(_gather_body, npw, num_cores)
  return pl.kernel(body, out_type=out_t, mesh=mesh, scratch_types=scratch)(
      users2, pos2, neg2, u4, ipk)


def _score_body(gu, gp, gn, ps_ref, ns_ref, pl_ref, nl_ref):
  f32 = jnp.float32
  g = gu[...]
  unlo = lambda w: lax.bitcast_convert_type(w << 16, f32)
  unhi = lambda w: lax.bitcast_convert_type(w & jnp.uint32(0xFFFF0000), f32)
  server = unlo(g[:, 0:E])
  d0 = unhi(g[:, 0:E])
  d1 = unlo(g[:, E:2 * E])
  d2 = unhi(g[:, E:2 * E])
  eu4 = jnp.concatenate([server, d0, d1, d2], axis=1)
  ep = gp[:, 0:E]
  en = gn[:, 0:E]
  ep4 = jnp.concatenate([ep, ep, ep, ep], axis=1)
  en4 = jnp.concatenate([en, en, en, en], axis=1)
  ci = lax.broadcasted_iota(jnp.int32, (4 * E, 4), 0) // E
  ki = lax.broadcasted_iota(jnp.int32, (4 * E, 4), 1)
  m2 = (ci <= ki).astype(f32)
  pcum = lax.dot_general(m2, eu4 * ep4, (((0,), (1,)), ((), ())),
                         preferred_element_type=f32)
  ncum = lax.dot_general(m2, eu4 * en4, (((0,), (1,)), ((), ())),
                         preferred_element_type=f32)
  ps_ref[...] = pcum[3]
  ns_ref[...] = ncum[3]
  pl_ref[...] = pcum
  nl_ref[...] = ncum


def _score(gu, gp, gn):
  grid = (B // SBLK,)
  return pl.pallas_call(
      _score_body,
      grid=grid,
      in_specs=[
          pl.BlockSpec((SBLK, 2 * E), lambda i: (i, 0)),
          pl.BlockSpec((SBLK, 2 * E), lambda i: (i, 0)),
          pl.BlockSpec((SBLK, 2 * E), lambda i: (i, 0)),
      ],
      out_specs=[
          pl.BlockSpec((SBLK,), lambda i: (i,)),
          pl.BlockSpec((SBLK,), lambda i: (i,)),
          pl.BlockSpec((4, SBLK), lambda i: (0, i)),
          pl.BlockSpec((4, SBLK), lambda i: (0, i)),
      ],
      out_shape=[
          jax.ShapeDtypeStruct((B,), jnp.float32),
          jax.ShapeDtypeStruct((B,), jnp.float32),
          jax.ShapeDtypeStruct((4, B), jnp.float32),
          jax.ShapeDtypeStruct((4, B), jnp.float32),
      ],
  )(gu, gp, gn)


def kernel(users, pos_items, neg_items, W_user, W_item, C0, C1, C2,
           W_dnn, b_dnn, W_di, b_di, Wd0, bd0, Wd1, bd1, Wd2, bd2):
  u4, ipk = _transform(W_user, W_item, C0, C1, C2, W_dnn, b_dnn,
                       W_di, b_di, Wd0, bd0, Wd1, bd1, Wd2, bd2)
  gu, gp, gn = _sc_gather(users, pos_items, neg_items, u4, ipk)
  return _score(gu, gp, gn)


# speedup vs baseline: 1.1076x; 1.1076x over previous
import functools

import jax
import jax.numpy as jnp
from jax import lax
from jax.experimental import pallas as pl
from jax.experimental.pallas import tpu as pltpu
from jax.experimental.pallas import tpu_sc as plsc

U = 100000
I = 100000
E = 64
B = 16384

CHUNK = 128
NUM_CHUNKS = B // CHUNK
TBLK = 4096
SBLK = 2048

_TLHS = (((0,), (0,)), ((), ()))
_TLHS_RT = (((0,), (1,)), ((), ()))


def _user_body(wut, c0t, c1t, c2t, wit, w_dnn_t, b_dnn, w_di, b_di,
               wd0, bd0, wd1, bd1, wd2, bd2, u4_ref, ipk_ref):
  f32 = jnp.float32
  ucat_t = jnp.concatenate(
      [wut[...], c0t[...], c1t[...], c2t[...]], axis=0)
  server = jax.nn.relu(
      lax.dot_general(ucat_t, w_dnn_t[...], _TLHS_RT,
                      preferred_element_type=f32) + b_dnn[...])
  d0 = jax.nn.relu(
      lax.dot_general(c0t[...], wd0[...], _TLHS,
                      preferred_element_type=f32) + bd0[...])
  d1 = jax.nn.relu(
      lax.dot_general(c1t[...], wd1[...], _TLHS,
                      preferred_element_type=f32) + bd1[...])
  d2 = jax.nn.relu(
      lax.dot_general(c2t[...], wd2[...], _TLHS,
                      preferred_element_type=f32) + bd2[...])

  def pack(a, b):
    a32 = lax.bitcast_convert_type(
        a.astype(jnp.bfloat16), jnp.uint16).astype(jnp.uint32)
    b32 = lax.bitcast_convert_type(
        b.astype(jnp.bfloat16), jnp.uint16).astype(jnp.uint32)
    return a32 | (b32 << 16)

  u4_ref[:, 0:E] = pack(server, d0)
  u4_ref[:, E:2 * E] = pack(d1, d2)
  e = jax.nn.relu(
      lax.dot_general(wit[...], w_di[...], _TLHS,
                      preferred_element_type=f32) + b_di[...])
  ipk_ref[:, 0:E] = e
  ipk_ref[:, E:2 * E] = e


def _transform(w_user, w_item, c0, c1, c2, w_dnn, b_dnn, w_di, b_di,
               wd0, bd0, wd1, bd1, wd2, bd2):
  grid = (pl.cdiv(U, TBLK),)
  t_spec = pl.BlockSpec((E, TBLK), lambda i: (0, i))
  full = lambda shape: pl.BlockSpec(shape, lambda i: (0,) * len(shape))
  return pl.pallas_call(
      _user_body,
      grid=grid,
      in_specs=[t_spec] * 5 + [
          full((E, 4 * E)), full((1, E)),
          full((E, E)), full((1, E)),
          full((E, E)), full((1, E)),
          full((E, E)), full((1, E)),
          full((E, E)), full((1, E)),
      ],
      out_specs=[
          pl.BlockSpec((TBLK, 2 * E), lambda i: (i, 0)),
          pl.BlockSpec((TBLK, 2 * E), lambda i: (i, 0)),
      ],
      out_shape=[
          jax.ShapeDtypeStruct((U, 2 * E), jnp.uint32),
          jax.ShapeDtypeStruct((I, 2 * E), jnp.float32),
      ],
      compiler_params=pltpu.CompilerParams(
          fuse_transposed_lhs_in_matmul=True),
  )(w_user.T, c0.T, c1.T, c2.T, w_item.T, w_dnn.T, b_dnn.reshape(1, E),
    w_di, b_di.reshape(1, E), wd0, bd0.reshape(1, E),
    wd1, bd1.reshape(1, E), wd2, bd2.reshape(1, E))


def _gather_body(nchunks_per_worker, num_cores,
                 users_hbm, pos_hbm, neg_hbm, u4, ipk,
                 out_u, out_p, out_n,
                 idx_v, ub0, ub1, ib0, ib1, gsem, ssem):
  wid = lax.axis_index("s") * num_cores + lax.axis_index("c")
  row0 = wid * nchunks_per_worker
  n = nchunks_per_worker
  pltpu.sync_copy(users_hbm.at[pl.ds(row0, n)], idx_v.at[pl.ds(0, n)])
  pltpu.sync_copy(pos_hbm.at[pl.ds(row0, n)], idx_v.at[pl.ds(n, n)])
  pltpu.sync_copy(neg_hbm.at[pl.ds(row0, n)], idx_v.at[pl.ds(2 * n, n)])
  work = []
  for j in range(n):
    base = (row0 + j) * CHUNK
    work.append((u4, j, out_u, base, (ub0, ub1)[j % 2]))
    work.append((ipk, n + j, out_p, base, ib0))
    work.append((ipk, 2 * n + j, out_n, base, ib1))
  pending = {}
  for table, irow, out, base, buf in work:
    if id(buf) in pending:
      pending.pop(id(buf)).wait()
    pltpu.async_copy(table.at[idx_v.at[irow]], buf, gsem).wait()
    pending[id(buf)] = pltpu.async_copy(
        buf, out.at[pl.ds(base, CHUNK)], ssem)
  for h in pending.values():
    h.wait()


def _sc_gather(users, pos_items, neg_items, u4, ipk):
  info = plsc.get_sparse_core_info()
  num_cores, num_subcores = info.num_cores, info.num_subcores
  npw = NUM_CHUNKS // (num_cores * num_subcores)
  mesh = plsc.VectorSubcoreMesh(core_axis_name="c", subcore_axis_name="s")
  out_t = [
      jax.ShapeDtypeStruct((B, 2 * E), jnp.uint32),
      jax.ShapeDtypeStruct((B, 2 * E), jnp.float32),
      jax.ShapeDtypeStruct((B, 2 * E), jnp.float32),
  ]
  scratch = [
      pltpu.VMEM((3 * npw, CHUNK), jnp.int32),
      pltpu.VMEM((CHUNK, 2 * E), jnp.uint32),
      pltpu.VMEM((CHUNK, 2 * E), jnp.uint32),
      pltpu.VMEM((CHUNK, 2 * E), jnp.float32),
      pltpu.VMEM((CHUNK, 2 * E), jnp.float32),
      pltpu.SemaphoreType.DMA,
      pltpu.SemaphoreType.DMA,
  ]
  users2 = users.astype(jnp.int32).reshape(NUM_CHUNKS, CHUNK)
  pos2 = pos_items.astype(jnp.int32).reshape(NUM_CHUNKS, CHUNK)
  neg2 = neg_items.astype(jnp.int32).reshape(NUM_CHUNKS, CHUNK)
  body = functools.partial(_gather_body, npw, num_cores)
  return pl.kernel(body, out_type=out_t, mesh=mesh, scratch_types=scratch)(
      users2, pos2, neg2, u4, ipk)


def _score_body(gu, gp, gn, ps_ref, ns_ref, pl_ref, nl_ref):
  f32 = jnp.float32
  g = gu[...]
  unlo = lambda w: lax.bitcast_convert_type(w << 16, f32)
  unhi = lambda w: lax.bitcast_convert_type(w & jnp.uint32(0xFFFF0000), f32)
  server = unlo(g[:, 0:E])
  d0 = unhi(g[:, 0:E])
  d1 = unlo(g[:, E:2 * E])
  d2 = unhi(g[:, E:2 * E])
  eu4 = jnp.concatenate([server, d0, d1, d2], axis=1)
  ep = gp[:, 0:E]
  en = gn[:, 0:E]
  ep4 = jnp.concatenate([ep, ep, ep, ep], axis=1)
  en4 = jnp.concatenate([en, en, en, en], axis=1)
  ci = lax.broadcasted_iota(jnp.int32, (4 * E, 4), 0) // E
  ki = lax.broadcasted_iota(jnp.int32, (4 * E, 4), 1)
  m2 = (ci <= ki).astype(f32)
  pcum = lax.dot_general(m2, eu4 * ep4, (((0,), (1,)), ((), ())),
                         preferred_element_type=f32)
  ncum = lax.dot_general(m2, eu4 * en4, (((0,), (1,)), ((), ())),
                         preferred_element_type=f32)
  ps_ref[...] = pcum[3]
  ns_ref[...] = ncum[3]
  pl_ref[...] = pcum
  nl_ref[...] = ncum


def _score(gu, gp, gn):
  grid = (B // SBLK,)
  return pl.pallas_call(
      _score_body,
      grid=grid,
      in_specs=[
          pl.BlockSpec((SBLK, 2 * E), lambda i: (i, 0)),
          pl.BlockSpec((SBLK, 2 * E), lambda i: (i, 0)),
          pl.BlockSpec((SBLK, 2 * E), lambda i: (i, 0)),
      ],
      out_specs=[
          pl.BlockSpec((SBLK,), lambda i: (i,)),
          pl.BlockSpec((SBLK,), lambda i: (i,)),
          pl.BlockSpec((4, SBLK), lambda i: (0, i)),
          pl.BlockSpec((4, SBLK), lambda i: (0, i)),
      ],
      out_shape=[
          jax.ShapeDtypeStruct((B,), jnp.float32),
          jax.ShapeDtypeStruct((B,), jnp.float32),
          jax.ShapeDtypeStruct((4, B), jnp.float32),
          jax.ShapeDtypeStruct((4, B), jnp.float32),
      ],
  )(gu, gp, gn)


def kernel(users, pos_items, neg_items, W_user, W_item, C0, C1, C2,
           W_dnn, b_dnn, W_di, b_di, Wd0, bd0, Wd1, bd1, Wd2, bd2):
  u4, ipk = _transform(W_user, W_item, C0, C1, C2, W_dnn, b_dnn,
                       W_di, b_di, Wd0, bd0, Wd1, bd1, Wd2, bd2)
  gu, gp, gn = _sc_gather(users, pos_items, neg_items, u4, ipk)
  return _score(gu, gp, gn)
